# baseline (speedup 1.0000x reference)
import functools

import jax
import jax.numpy as jnp
from jax import lax
from jax.experimental import pallas as pl
from jax.experimental.pallas import tpu as pltpu
from jax.experimental.pallas import tpu_sc as plsc

N = 10000
E = 320000
D = 128

NC = 2
NS = 16
NW = NC * NS
CHUNK = 128
NCH = 80
NCHA = NCH + 4
EPTA = NCHA * CHUNK
ESC = NW * NCH * CHUNK
EPAD = NW * EPTA
NPAD = 10112
RPT = NPAD // NS

_MESH = plsc.VectorSubcoreMesh(core_axis_name="c", subcore_axis_name="s")

_ZOFFS = [0, 128, 256, 384, RPT - CHUNK]


def _sc_agg_body(with_deg, *refs):
    if with_deg:
        (y, srcp, dstp, zrow, zdeg, onesd, out_acc, out_deg, acc_sh, deg_sh,
         rwa, rwb, sx0, sx1, sx2, sx3, dx0, dx1, dx2, dx3, ones_v,
         ga, gb, sa, sb, da, db, i0, i1, i2, i3) = refs
        dsem = [da, db]
    else:
        (y, srcp, dstp, zrow, out_acc, acc_sh,
         rwa, rwb, sx0, sx1, sx2, sx3, dx0, dx1, dx2, dx3,
         ga, gb, sa, sb, i0, i1, i2, i3) = refs
    rows = [rwa, rwb]
    sidx = [sx0, sx1, sx2, sx3]
    didx = [dx0, dx1, dx2, dx3]
    gsem = [ga, gb]
    ssem = [sa, sb]
    isem = [i0, i1, i2, i3]

    c = lax.axis_index("c")
    s = lax.axis_index("s")
    wid = c * NS + s
    r0 = s * RPT
    ebase = wid * EPTA

    def idx_fill(q, ch):
        off = ebase + ch * CHUNK
        return (pltpu.async_copy(srcp.at[pl.ds(off, CHUNK)], sidx[q], isem[q]),
                pltpu.async_copy(dstp.at[pl.ds(off, CHUNK)], didx[q], isem[q]))

    iq = [idx_fill(q, q) for q in range(4)]
    iq[0][0].wait()
    iq[0][1].wait()
    gd = pltpu.async_copy(y.at[sidx[0]], rows[0], gsem[0])

    pltpu.sync_copy(zrow, rows[1])
    for o in _ZOFFS:
        pltpu.sync_copy(rows[1], acc_sh.at[pl.ds(r0 + o, CHUNK)])
    if with_deg:
        pltpu.sync_copy(zdeg, ones_v)
        for o in _ZOFFS:
            pltpu.sync_copy(ones_v, deg_sh.at[pl.ds(r0 + o, CHUNK)])
        pltpu.sync_copy(onesd, ones_v)
    plsc.subcore_barrier()

    for i in range(NCH):
        b = i % 2
        q = i % 4
        qn = (i + 1) % 4
        gd.wait()
        sd = pltpu.async_copy(rows[b], acc_sh.at[didx[q]], ssem[b], add=True)
        if with_deg:
            dd = pltpu.async_copy(ones_v, deg_sh.at[didx[q]], dsem[b],
                                  add=True)
        iq[qn][0].wait()
        iq[qn][1].wait()
        gd = pltpu.async_copy(y.at[sidx[qn]], rows[1 - b], gsem[1 - b])
        sd.wait()
        if with_deg:
            dd.wait()
        iq[q] = idx_fill(q, i + 4)
    gd.wait()
    for q in range(1, 4):
        iq[q][0].wait()
        iq[q][1].wait()
    plsc.subcore_barrier()

    ob = c * NPAD + r0
    for o in _ZOFFS:
        pltpu.sync_copy(acc_sh.at[pl.ds(r0 + o, CHUNK)], rows[0])
        pltpu.sync_copy(rows[0], out_acc.at[pl.ds(ob + o, CHUNK)])
    if with_deg:
        for o in _ZOFFS:
            pltpu.sync_copy(deg_sh.at[pl.ds(r0 + o, CHUNK)], ones_v)
            pltpu.sync_copy(ones_v, out_deg.at[pl.ds(ob + o, CHUNK)])


_sc_agg_deg = functools.partial(
    pl.kernel,
    functools.partial(_sc_agg_body, True),
    out_type=[
        jax.ShapeDtypeStruct((NC * NPAD, D), jnp.float32),
        jax.ShapeDtypeStruct((NC * NPAD,), jnp.float32),
    ],
    mesh=_MESH,
    scratch_types=[
        pltpu.VMEM_SHARED((NPAD, D), jnp.float32),
        pltpu.VMEM_SHARED((NPAD,), jnp.float32),
    ] + [pltpu.VMEM((CHUNK, D), jnp.float32)] * 2
      + [pltpu.VMEM((CHUNK,), jnp.int32)] * 8 + [
        pltpu.VMEM((CHUNK,), jnp.float32),
    ] + [pltpu.SemaphoreType.DMA] * 10,
)()

_sc_agg = functools.partial(
    pl.kernel,
    functools.partial(_sc_agg_body, False),
    out_type=jax.ShapeDtypeStruct((NC * NPAD, D), jnp.float32),
    mesh=_MESH,
    scratch_types=[
        pltpu.VMEM_SHARED((NPAD, D), jnp.float32),
    ] + [pltpu.VMEM((CHUNK, D), jnp.float32)] * 2
      + [pltpu.VMEM((CHUNK,), jnp.int32)] * 8
      + [pltpu.SemaphoreType.DMA] * 8,
)()


BM = 2000


def _tc_in_body(x_ref, ws_ref, wn_ref, b_ref, z_ref, y_ref):
    x = x_ref[...]
    z_ref[...] = (
        jnp.dot(x, ws_ref[...], preferred_element_type=jnp.float32) + b_ref[...]
    )
    y_ref[...] = jnp.dot(x, wn_ref[...], preferred_element_type=jnp.float32)


def _tc_mid_body(z1_ref, acc_ref, dg0_ref, dg1_ref, ws_ref, wn_ref, b_ref,
                 z2_ref, y2_ref):
    agg = acc_ref[0] + acc_ref[1]
    deg = jnp.maximum(dg0_ref[...] + dg1_ref[...], 1.0)
    h = jnp.maximum(z1_ref[...] + agg / deg, 0.0)
    z2_ref[...] = (
        jnp.dot(h, ws_ref[...], preferred_element_type=jnp.float32) + b_ref[...]
    )
    y2_ref[...] = jnp.dot(h, wn_ref[...], preferred_element_type=jnp.float32)


def _tc_out_body(z2_ref, acc_ref, dg0_ref, dg1_ref, out_ref):
    agg = acc_ref[0] + acc_ref[1]
    deg = jnp.maximum(dg0_ref[...] + dg1_ref[...], 1.0)
    out_ref[...] = z2_ref[...] + agg / deg


_row_spec = pl.BlockSpec((BM, D), lambda i: (i, 0))
_acc_spec = pl.BlockSpec((NC, BM, D), lambda i: (0, i, 0))
_deg_spec = pl.BlockSpec((BM, 1), lambda i: (i, 0))
_w_spec = pl.BlockSpec((D, D), lambda i: (0, 0))
_b_spec = pl.BlockSpec((1, D), lambda i: (0, 0))

_tc_in = pl.pallas_call(
    _tc_in_body,
    grid=(N // BM,),
    in_specs=[_row_spec, _w_spec, _w_spec, _b_spec],
    out_specs=[_row_spec, _row_spec],
    out_shape=[
        jax.ShapeDtypeStruct((N, D), jnp.float32),
        jax.ShapeDtypeStruct((N, D), jnp.float32),
    ],
)

_tc_mid = pl.pallas_call(
    _tc_mid_body,
    grid=(N // BM,),
    in_specs=[_row_spec, _acc_spec, _deg_spec, _deg_spec, _w_spec, _w_spec,
              _b_spec],
    out_specs=[_row_spec, _row_spec],
    out_shape=[
        jax.ShapeDtypeStruct((N, D), jnp.float32),
        jax.ShapeDtypeStruct((N, D), jnp.float32),
    ],
)

_tc_out = pl.pallas_call(
    _tc_out_body,
    grid=(N // BM,),
    in_specs=[_row_spec, _acc_spec, _deg_spec, _deg_spec],
    out_specs=_row_spec,
    out_shape=jax.ShapeDtypeStruct((N, D), jnp.float32),
)


@jax.jit
def kernel(x, edge_index, W1_self, W1_neigh, b1, W2_self, W2_neigh, b2):
    src = edge_index[0]
    dst = edge_index[1]
    srcp = jnp.concatenate(
        [jnp.pad(src, (0, ESC - E)).reshape(NW, NCH * CHUNK),
         jnp.zeros((NW, (NCHA - NCH) * CHUNK), jnp.int32)], axis=1
    ).reshape(-1)
    dstp = jnp.concatenate(
        [jnp.pad(dst, (0, ESC - E), constant_values=N).reshape(NW, NCH * CHUNK),
         jnp.full((NW, (NCHA - NCH) * CHUNK), N, jnp.int32)], axis=1
    ).reshape(-1)
    zrow = jnp.zeros((CHUNK, D), jnp.float32)
    zdeg = jnp.zeros((CHUNK,), jnp.float32)
    onesd = jnp.ones((CHUNK,), jnp.float32)

    z1, y1 = _tc_in(x, W1_self, W1_neigh, b1.reshape(1, D))
    acc1, deg = _sc_agg_deg(y1, srcp, dstp, zrow, zdeg, onesd)
    acc1 = acc1.reshape(NC, NPAD, D)[:, :N]
    degn = deg.reshape(NC, NPAD)[:, :N]
    dg0 = degn[0][:, None]
    dg1 = degn[1][:, None]
    z2, y2 = _tc_mid(z1, acc1, dg0, dg1, W2_self, W2_neigh, b2.reshape(1, D))
    acc2 = _sc_agg(y2, srcp, dstp, zrow)
    return _tc_out(z2, acc2.reshape(NC, NPAD, D)[:, :N], dg0, dg1)


# speedup vs baseline: 1.1441x; 1.1441x over previous
import functools

import jax
import jax.numpy as jnp
from jax import lax
from jax.experimental import pallas as pl
from jax.experimental.pallas import tpu as pltpu
from jax.experimental.pallas import tpu_sc as plsc

N = 10000
E = 320000
D = 128

NC = 2
NS = 16
NW = NC * NS
CHUNK = 128
NCH = 80
NCHA = NCH + 4
EPTA = NCHA * CHUNK
ESC = NW * NCH * CHUNK
EPAD = NW * EPTA
NPAD = 10112
RPT = NPAD // NS

_MESH = plsc.VectorSubcoreMesh(core_axis_name="c", subcore_axis_name="s")

_ZOFFS = [0, 128, 256, 384, RPT - CHUNK]


def _sc_agg_body(with_deg, *refs):
    if with_deg:
        (y, srcp, dstp, zrow, zdeg, onesd, out_acc, out_deg, acc_sh, deg_sh,
         rwa, sidx4, didx4, ones_v, ga) = refs
    else:
        (y, srcp, dstp, zrow, out_acc, acc_sh,
         rwa, sidx4, didx4, ga) = refs
    rows = [rwa]
    gsem = [ga]

    c = lax.axis_index("c")
    s = lax.axis_index("s")
    wid = c * NS + s
    r0 = s * RPT

    pltpu.sync_copy(zrow, rows[0])
    for o in _ZOFFS:
        pltpu.sync_copy(rows[0], acc_sh.at[pl.ds(r0 + o, CHUNK)])
    if with_deg:
        pltpu.sync_copy(zdeg, ones_v)
        for o in _ZOFFS:
            pltpu.sync_copy(ones_v, deg_sh.at[pl.ds(r0 + o, CHUNK)])
        pltpu.sync_copy(onesd, ones_v)
    plsc.subcore_barrier()

    def group_body(g, carry):
        roff = wid * NCH + g * 8
        pltpu.sync_copy(srcp.at[pl.ds(roff, 8)], sidx4)
        pltpu.sync_copy(dstp.at[pl.ds(roff, 8)], didx4)
        for j in range(8):
            pltpu.async_copy(y.at[sidx4.at[j]], rows[0], gsem[0]).wait()
            pltpu.sync_copy(rows[0], acc_sh.at[didx4.at[j]], add=True)
            if with_deg:
                pltpu.sync_copy(ones_v, deg_sh.at[didx4.at[j]], add=True)
        return carry

    lax.fori_loop(0, NCH // 8, group_body, 0)
    plsc.subcore_barrier()

    ob = c * NPAD + r0
    for o in _ZOFFS:
        pltpu.sync_copy(acc_sh.at[pl.ds(r0 + o, CHUNK)], rows[0])
        pltpu.sync_copy(rows[0], out_acc.at[pl.ds(ob + o, CHUNK)])
    if with_deg:
        for o in _ZOFFS:
            pltpu.sync_copy(deg_sh.at[pl.ds(r0 + o, CHUNK)], ones_v)
            pltpu.sync_copy(ones_v, out_deg.at[pl.ds(ob + o, CHUNK)])


_sc_agg_deg = functools.partial(
    pl.kernel,
    functools.partial(_sc_agg_body, True),
    out_type=[
        jax.ShapeDtypeStruct((NC * NPAD, D), jnp.float32),
        jax.ShapeDtypeStruct((NC * NPAD,), jnp.float32),
    ],
    mesh=_MESH,
    scratch_types=[
        pltpu.VMEM_SHARED((NPAD, D), jnp.float32),
        pltpu.VMEM_SHARED((NPAD,), jnp.float32),
    ] + [pltpu.VMEM((CHUNK, D), jnp.float32)] * 1 + [
        pltpu.VMEM((8, CHUNK), jnp.int32),
        pltpu.VMEM((8, CHUNK), jnp.int32),
        pltpu.VMEM((CHUNK,), jnp.float32),
    ] + [pltpu.SemaphoreType.DMA] * 1,
)()

_sc_agg = functools.partial(
    pl.kernel,
    functools.partial(_sc_agg_body, False),
    out_type=jax.ShapeDtypeStruct((NC * NPAD, D), jnp.float32),
    mesh=_MESH,
    scratch_types=[
        pltpu.VMEM_SHARED((NPAD, D), jnp.float32),
    ] + [pltpu.VMEM((CHUNK, D), jnp.float32)] * 1 + [
        pltpu.VMEM((8, CHUNK), jnp.int32),
        pltpu.VMEM((8, CHUNK), jnp.int32),
    ] + [pltpu.SemaphoreType.DMA] * 1,
)()


BM = 2000


def _tc_in_body(x_ref, ws_ref, wn_ref, b_ref, z_ref, y_ref):
    x = x_ref[...]
    z_ref[...] = (
        jnp.dot(x, ws_ref[...], preferred_element_type=jnp.float32) + b_ref[...]
    )
    y_ref[...] = jnp.dot(x, wn_ref[...], preferred_element_type=jnp.float32)


def _tc_mid_body(z1_ref, acc_ref, dg0_ref, dg1_ref, ws_ref, wn_ref, b_ref,
                 z2_ref, y2_ref):
    agg = acc_ref[0] + acc_ref[1]
    deg = jnp.maximum(dg0_ref[...] + dg1_ref[...], 1.0)
    h = jnp.maximum(z1_ref[...] + agg / deg, 0.0)
    z2_ref[...] = (
        jnp.dot(h, ws_ref[...], preferred_element_type=jnp.float32) + b_ref[...]
    )
    y2_ref[...] = jnp.dot(h, wn_ref[...], preferred_element_type=jnp.float32)


def _tc_out_body(z2_ref, acc_ref, dg0_ref, dg1_ref, out_ref):
    agg = acc_ref[0] + acc_ref[1]
    deg = jnp.maximum(dg0_ref[...] + dg1_ref[...], 1.0)
    out_ref[...] = z2_ref[...] + agg / deg


_row_spec = pl.BlockSpec((BM, D), lambda i: (i, 0))
_acc_spec = pl.BlockSpec((NC, BM, D), lambda i: (0, i, 0))
_deg_spec = pl.BlockSpec((BM, 1), lambda i: (i, 0))
_w_spec = pl.BlockSpec((D, D), lambda i: (0, 0))
_b_spec = pl.BlockSpec((1, D), lambda i: (0, 0))

_tc_in = pl.pallas_call(
    _tc_in_body,
    grid=(N // BM,),
    in_specs=[_row_spec, _w_spec, _w_spec, _b_spec],
    out_specs=[_row_spec, _row_spec],
    out_shape=[
        jax.ShapeDtypeStruct((N, D), jnp.float32),
        jax.ShapeDtypeStruct((N, D), jnp.float32),
    ],
)

_tc_mid = pl.pallas_call(
    _tc_mid_body,
    grid=(N // BM,),
    in_specs=[_row_spec, _acc_spec, _deg_spec, _deg_spec, _w_spec, _w_spec,
              _b_spec],
    out_specs=[_row_spec, _row_spec],
    out_shape=[
        jax.ShapeDtypeStruct((N, D), jnp.float32),
        jax.ShapeDtypeStruct((N, D), jnp.float32),
    ],
)

_tc_out = pl.pallas_call(
    _tc_out_body,
    grid=(N // BM,),
    in_specs=[_row_spec, _acc_spec, _deg_spec, _deg_spec],
    out_specs=_row_spec,
    out_shape=jax.ShapeDtypeStruct((N, D), jnp.float32),
)


@jax.jit
def kernel(x, edge_index, W1_self, W1_neigh, b1, W2_self, W2_neigh, b2):
    src = edge_index[0]
    dst = edge_index[1]
    srcp = jnp.pad(src, (0, ESC - E)).reshape(NW * NCH, CHUNK)
    dstp = jnp.pad(dst, (0, ESC - E), constant_values=N).reshape(
        NW * NCH, CHUNK)
    zrow = jnp.zeros((CHUNK, D), jnp.float32)
    zdeg = jnp.zeros((CHUNK,), jnp.float32)
    onesd = jnp.ones((CHUNK,), jnp.float32)

    z1, y1 = _tc_in(x, W1_self, W1_neigh, b1.reshape(1, D))
    acc1, deg = _sc_agg_deg(y1, srcp, dstp, zrow, zdeg, onesd)
    acc1 = acc1.reshape(NC, NPAD, D)[:, :N]
    degn = deg.reshape(NC, NPAD)[:, :N]
    dg0 = degn[0][:, None]
    dg1 = degn[1][:, None]
    z2, y2 = _tc_mid(z1, acc1, dg0, dg1, W2_self, W2_neigh, b2.reshape(1, D))
    acc2 = _sc_agg(y2, srcp, dstp, zrow)
    return _tc_out(z2, acc2.reshape(NC, NPAD, D)[:, :N], dg0, dg1)


# speedup vs baseline: 1.3316x; 1.1639x over previous
import functools

import jax
import jax.numpy as jnp
from jax import lax
from jax.experimental import pallas as pl
from jax.experimental.pallas import tpu as pltpu
from jax.experimental.pallas import tpu_sc as plsc

N = 10000
E = 320000
D = 128

NC = 2
NS = 16
NW = NC * NS
CHUNK = 128
NCH = 80
NCHA = NCH + 4
EPTA = NCHA * CHUNK
ESC = NW * NCH * CHUNK
EPAD = NW * EPTA
NPAD = 10112
RPT = NPAD // NS

_MESH = plsc.VectorSubcoreMesh(core_axis_name="c", subcore_axis_name="s")

_ZOFFS = [0, 128, 256, 384, RPT - CHUNK]


def _sc_agg_body(with_deg, *refs):
    if with_deg:
        (y, srcp, dstp, zrow, zdeg, onesd, out_acc, out_deg, acc_sh, deg_sh,
         rows, sxa, sxb, dxa, dxb, ones_v, gsem, ia, ib) = refs
    else:
        (y, srcp, dstp, zrow, out_acc, acc_sh,
         rows, sxa, sxb, dxa, dxb, gsem, ia, ib) = refs
    sidx = [sxa, sxb]
    didx = [dxa, dxb]
    isem = [ia, ib]

    c = lax.axis_index("c")
    s = lax.axis_index("s")
    wid = c * NS + s
    r0 = s * RPT
    ebase = wid * NCH * CHUNK

    def idx_fill(p, ch):
        off = ebase + ch * CHUNK
        return (pltpu.async_copy(srcp.at[pl.ds(off, CHUNK)], sidx[p], isem[p]),
                pltpu.async_copy(dstp.at[pl.ds(off, CHUNK)], didx[p], isem[p]))

    iq = idx_fill(0, 0)
    iq[0].wait()
    iq[1].wait()

    pltpu.sync_copy(zrow, rows)
    for o in _ZOFFS:
        pltpu.sync_copy(rows, acc_sh.at[pl.ds(r0 + o, CHUNK)])
    if with_deg:
        pltpu.sync_copy(zdeg, ones_v)
        for o in _ZOFFS:
            pltpu.sync_copy(ones_v, deg_sh.at[pl.ds(r0 + o, CHUNK)])
        pltpu.sync_copy(onesd, ones_v)
    plsc.subcore_barrier()

    for i in range(NCH):
        p = i % 2
        gd = pltpu.async_copy(y.at[sidx[p]], rows, gsem)
        if i + 1 < NCH:
            iq = idx_fill(1 - p, i + 1)
        gd.wait()
        pltpu.sync_copy(rows, acc_sh.at[didx[p]], add=True)
        if with_deg:
            pltpu.sync_copy(ones_v, deg_sh.at[didx[p]], add=True)
        if i + 1 < NCH:
            iq[0].wait()
            iq[1].wait()
    plsc.subcore_barrier()

    ob = c * NPAD + r0
    for o in _ZOFFS:
        pltpu.sync_copy(acc_sh.at[pl.ds(r0 + o, CHUNK)], rows)
        pltpu.sync_copy(rows, out_acc.at[pl.ds(ob + o, CHUNK)])
    if with_deg:
        for o in _ZOFFS:
            pltpu.sync_copy(deg_sh.at[pl.ds(r0 + o, CHUNK)], ones_v)
            pltpu.sync_copy(ones_v, out_deg.at[pl.ds(ob + o, CHUNK)])


_sc_agg_deg = functools.partial(
    pl.kernel,
    functools.partial(_sc_agg_body, True),
    out_type=[
        jax.ShapeDtypeStruct((NC * NPAD, D), jnp.float32),
        jax.ShapeDtypeStruct((NC * NPAD,), jnp.float32),
    ],
    mesh=_MESH,
    scratch_types=[
        pltpu.VMEM_SHARED((NPAD, D), jnp.float32),
        pltpu.VMEM_SHARED((NPAD,), jnp.float32),
        pltpu.VMEM((CHUNK, D), jnp.float32),
    ] + [pltpu.VMEM((CHUNK,), jnp.int32)] * 4 + [
        pltpu.VMEM((CHUNK,), jnp.float32),
    ] + [pltpu.SemaphoreType.DMA] * 3,
)()

_sc_agg = functools.partial(
    pl.kernel,
    functools.partial(_sc_agg_body, False),
    out_type=jax.ShapeDtypeStruct((NC * NPAD, D), jnp.float32),
    mesh=_MESH,
    scratch_types=[
        pltpu.VMEM_SHARED((NPAD, D), jnp.float32),
        pltpu.VMEM((CHUNK, D), jnp.float32),
    ] + [pltpu.VMEM((CHUNK,), jnp.int32)] * 4
      + [pltpu.SemaphoreType.DMA] * 3,
)()


BM = 2000


def _tc_in_body(x_ref, ws_ref, wn_ref, b_ref, z_ref, y_ref):
    x = x_ref[...]
    z_ref[...] = (
        jnp.dot(x, ws_ref[...], preferred_element_type=jnp.float32) + b_ref[...]
    )
    y_ref[...] = jnp.dot(x, wn_ref[...], preferred_element_type=jnp.float32)


def _tc_mid_body(z1_ref, acc_ref, dg0_ref, dg1_ref, ws_ref, wn_ref, b_ref,
                 z2_ref, y2_ref):
    agg = acc_ref[0] + acc_ref[1]
    deg = jnp.maximum(dg0_ref[...] + dg1_ref[...], 1.0)
    h = jnp.maximum(z1_ref[...] + agg / deg, 0.0)
    z2_ref[...] = (
        jnp.dot(h, ws_ref[...], preferred_element_type=jnp.float32) + b_ref[...]
    )
    y2_ref[...] = jnp.dot(h, wn_ref[...], preferred_element_type=jnp.float32)


def _tc_out_body(z2_ref, acc_ref, dg0_ref, dg1_ref, out_ref):
    agg = acc_ref[0] + acc_ref[1]
    deg = jnp.maximum(dg0_ref[...] + dg1_ref[...], 1.0)
    out_ref[...] = z2_ref[...] + agg / deg


_row_spec = pl.BlockSpec((BM, D), lambda i: (i, 0))
_acc_spec = pl.BlockSpec((NC, BM, D), lambda i: (0, i, 0))
_deg_spec = pl.BlockSpec((BM, 1), lambda i: (i, 0))
_w_spec = pl.BlockSpec((D, D), lambda i: (0, 0))
_b_spec = pl.BlockSpec((1, D), lambda i: (0, 0))

_tc_in = pl.pallas_call(
    _tc_in_body,
    grid=(N // BM,),
    in_specs=[_row_spec, _w_spec, _w_spec, _b_spec],
    out_specs=[_row_spec, _row_spec],
    out_shape=[
        jax.ShapeDtypeStruct((N, D), jnp.float32),
        jax.ShapeDtypeStruct((N, D), jnp.float32),
    ],
)

_tc_mid = pl.pallas_call(
    _tc_mid_body,
    grid=(N // BM,),
    in_specs=[_row_spec, _acc_spec, _deg_spec, _deg_spec, _w_spec, _w_spec,
              _b_spec],
    out_specs=[_row_spec, _row_spec],
    out_shape=[
        jax.ShapeDtypeStruct((N, D), jnp.float32),
        jax.ShapeDtypeStruct((N, D), jnp.float32),
    ],
)

_tc_out = pl.pallas_call(
    _tc_out_body,
    grid=(N // BM,),
    in_specs=[_row_spec, _acc_spec, _deg_spec, _deg_spec],
    out_specs=_row_spec,
    out_shape=jax.ShapeDtypeStruct((N, D), jnp.float32),
)


@jax.jit
def kernel(x, edge_index, W1_self, W1_neigh, b1, W2_self, W2_neigh, b2):
    src = edge_index[0]
    dst = edge_index[1]
    srcp = jnp.pad(src, (0, ESC - E))
    dstp = jnp.pad(dst, (0, ESC - E), constant_values=N)
    zrow = jnp.zeros((CHUNK, D), jnp.float32)
    zdeg = jnp.zeros((CHUNK,), jnp.float32)
    onesd = jnp.ones((CHUNK,), jnp.float32)

    z1, y1 = _tc_in(x, W1_self, W1_neigh, b1.reshape(1, D))
    acc1, deg = _sc_agg_deg(y1, srcp, dstp, zrow, zdeg, onesd)
    acc1 = acc1.reshape(NC, NPAD, D)[:, :N]
    degn = deg.reshape(NC, NPAD)[:, :N]
    dg0 = degn[0][:, None]
    dg1 = degn[1][:, None]
    z2, y2 = _tc_mid(z1, acc1, dg0, dg1, W2_self, W2_neigh, b2.reshape(1, D))
    acc2 = _sc_agg(y2, srcp, dstp, zrow)
    return _tc_out(z2, acc2.reshape(NC, NPAD, D)[:, :N], dg0, dg1)
